# Initial kernel scaffold; baseline (speedup 1.0000x reference)
import functools

import jax
import jax.numpy as jnp
from jax import lax
from jax.experimental import pallas as pl
from jax.experimental.pallas import tpu as pltpu
from jax.experimental.pallas import tpu_sc as plsc

N = 10000
E = 320000
D = 128
HC = 128
HEADS = 4
C = 32
NEG = 0.2

NC = 2
NS = 16
EPW = E // (NC * NS)
B = 80
NCHUNK = EPW // B
ROWW = HC + 16
RPT = N // NS
ZROWS = 125


def _transform_body(x_ref, wl_ref, bl_ref, wr_ref, br_ref, xl_ref, xr_ref):
    xb = x_ref[...]
    xl_ref[...] = jnp.dot(xb, wl_ref[...],
                          preferred_element_type=jnp.float32) + bl_ref[...]
    xr_ref[...] = jnp.dot(xb, wr_ref[...],
                          preferred_element_type=jnp.float32) + br_ref[...]


def _transform(x, W_l, b_l, W_r, b_r):
    R = 2000
    return pl.pallas_call(
        _transform_body,
        grid=(N // R,),
        in_specs=[
            pl.BlockSpec((R, D), lambda i: (i, 0)),
            pl.BlockSpec((D, HC), lambda i: (0, 0)),
            pl.BlockSpec((1, HC), lambda i: (0, 0)),
            pl.BlockSpec((D, HC), lambda i: (0, 0)),
            pl.BlockSpec((1, HC), lambda i: (0, 0)),
        ],
        out_specs=[
            pl.BlockSpec((R, HC), lambda i: (i, 0)),
            pl.BlockSpec((R, HC), lambda i: (i, 0)),
        ],
        out_shape=[
            jax.ShapeDtypeStruct((N, HC), jnp.float32),
            jax.ShapeDtypeStruct((N, HC), jnp.float32),
        ],
    )(x, W_l, b_l.reshape(1, HC), W_r, b_r.reshape(1, HC))


def _sc_edge_body(xl_hbm, xr_hbm, src_hbm, dst_hbm, att_hbm, out_hbm,
                  srcv, dstv, xlv, xrv, msgv, attv, zbuf, acc, sem1, sem2):
    cid = lax.axis_index("c")
    sid = lax.axis_index("s")

    zero16 = jnp.zeros((16,), jnp.float32)

    def zrow(r, carry):
        for k in range(ROWW // 16):
            zbuf[r, pl.ds(16 * k, 16)] = zero16
        return carry

    lax.fori_loop(0, ZROWS, zrow, 0)
    for j in range(RPT // ZROWS):
        pltpu.sync_copy(zbuf, acc.at[pl.ds(sid * RPT + j * ZROWS, ZROWS)])

    pltpu.sync_copy(att_hbm, attv)
    attregs = [attv[pl.ds(16 * k, 16)] for k in range(HC // 16)]
    lane = lax.broadcasted_iota(jnp.int32, (16,), 0)
    plsc.subcore_barrier()

    base_edge = (cid * NS + sid) * EPW

    def edge_body(e, carry):
        xl = [xlv[e, pl.ds(16 * k, 16)] for k in range(8)]
        xr = [xrv[e, pl.ds(16 * k, 16)] for k in range(8)]
        pv = jnp.zeros((16,), jnp.float32)
        for h in range(HEADS):
            s0 = xl[2 * h] + xr[2 * h]
            s1 = xl[2 * h + 1] + xr[2 * h + 1]
            t0 = jnp.maximum(s0, NEG * s0)
            t1 = jnp.maximum(s1, NEG * s1)
            u = t0 * attregs[2 * h] + t1 * attregs[2 * h + 1]
            lh = jnp.sum(u)
            ph = jnp.exp(jnp.full((16,), lh, jnp.float32))
            msgv[e, pl.ds(h * 32, 16)] = ph * xl[2 * h]
            msgv[e, pl.ds(h * 32 + 16, 16)] = ph * xl[2 * h + 1]
            pv = jnp.where(lane == h, ph, pv)
        msgv[e, pl.ds(HC, 16)] = pv
        return carry

    def chunk_body(i, carry):
        base = base_edge + i * B
        pltpu.sync_copy(src_hbm.at[pl.ds(base, B)], srcv)
        pltpu.sync_copy(dst_hbm.at[pl.ds(base, B)], dstv)
        cp1 = pltpu.async_copy(xl_hbm.at[srcv], xlv, sem1)
        cp2 = pltpu.async_copy(xr_hbm.at[dstv], xrv, sem2)
        cp1.wait()
        cp2.wait()
        lax.fori_loop(0, B, edge_body, 0)
        pltpu.sync_copy(msgv, acc.at[dstv], add=True)
        return carry

    lax.fori_loop(0, NCHUNK, chunk_body, 0)
    plsc.subcore_barrier()
    for j in range(RPT // ZROWS):
        r0 = sid * RPT + j * ZROWS
        pltpu.sync_copy(acc.at[pl.ds(r0, ZROWS)],
                        out_hbm.at[cid, pl.ds(r0, ZROWS)])


def _sc_edge(xl, xr, src, dst, att_flat):
    mesh = plsc.VectorSubcoreMesh(core_axis_name="c", subcore_axis_name="s")
    f = pl.kernel(
        _sc_edge_body,
        out_type=jax.ShapeDtypeStruct((NC, N, ROWW), jnp.float32),
        mesh=mesh,
        scratch_types=[
            pltpu.VMEM((B,), jnp.int32),
            pltpu.VMEM((B,), jnp.int32),
            pltpu.VMEM((B, D), jnp.float32),
            pltpu.VMEM((B, D), jnp.float32),
            pltpu.VMEM((B, ROWW), jnp.float32),
            pltpu.VMEM((HC,), jnp.float32),
            pltpu.VMEM((ZROWS, ROWW), jnp.float32),
            pltpu.VMEM_SHARED((N, ROWW), jnp.float32),
            pltpu.SemaphoreType.DMA,
            pltpu.SemaphoreType.DMA,
        ],
    )
    return f(xl, xr, src, dst, att_flat)


def _combine_body(p_ref, b_ref, o_ref):
    a = p_ref[0] + p_ref[1]
    num = a[:, :HC]
    rows = num.shape[0]
    li = lax.broadcasted_iota(jnp.int32, (rows, HC), 1)
    den = jnp.zeros((rows, HC), jnp.float32)
    for h in range(HEADS):
        dh = jnp.broadcast_to(a[:, HC + h:HC + h + 1], (rows, HC))
        den = jnp.where((li // C) == h, dh, den)
    o_ref[...] = num / (den + 1e-16) + b_ref[...]


def _combine(partials, bias):
    R = 2000
    return pl.pallas_call(
        _combine_body,
        grid=(N // R,),
        in_specs=[
            pl.BlockSpec((NC, R, ROWW), lambda i: (0, i, 0)),
            pl.BlockSpec((1, HC), lambda i: (0, 0)),
        ],
        out_specs=pl.BlockSpec((R, HC), lambda i: (i, 0)),
        out_shape=jax.ShapeDtypeStruct((N, HC), jnp.float32),
    )(partials, bias.reshape(1, HC))


def kernel(x, edge_index, W_l, b_l, W_r, b_r, att, bias):
    xl, xr = _transform(x, W_l, b_l, W_r, b_r)
    src = edge_index[0]
    dst = edge_index[1]
    partials = _sc_edge(xl, xr, src, dst, att.reshape(-1))
    return _combine(partials, bias)


# speedup vs baseline: 33.1503x; 33.1503x over previous
import jax
import jax.numpy as jnp
from jax import lax
from jax.experimental import pallas as pl
from jax.experimental.pallas import tpu as pltpu
from jax.experimental.pallas import tpu_sc as plsc

N = 10000
E = 320000
D = 128
HC = 128
HEADS = 4
C = 32
NEG = 0.2

NC = 2
NS = 16
EPW = E // (NC * NS)
B = 80
NCHUNK = EPW // B
N_PAD = 10240
RPT = N_PAD // NS
DROWS = N_PAD * HEADS // 128
ACC_ROWS = N_PAD + DROWS


def _transform_body(x_ref, wl_ref, bl_ref, wr_ref, br_ref, xl_ref, xr_ref):
    xb = x_ref[...]
    xl_ref[...] = jnp.dot(xb, wl_ref[...],
                          preferred_element_type=jnp.float32) + bl_ref[...]
    xr_ref[...] = jnp.dot(xb, wr_ref[...],
                          preferred_element_type=jnp.float32) + br_ref[...]


def _transform(x, W_l, b_l, W_r, b_r):
    R = 2000
    return pl.pallas_call(
        _transform_body,
        grid=(N // R,),
        in_specs=[
            pl.BlockSpec((R, D), lambda i: (i, 0)),
            pl.BlockSpec((D, HC), lambda i: (0, 0)),
            pl.BlockSpec((1, HC), lambda i: (0, 0)),
            pl.BlockSpec((D, HC), lambda i: (0, 0)),
            pl.BlockSpec((1, HC), lambda i: (0, 0)),
        ],
        out_specs=[
            pl.BlockSpec((R, HC), lambda i: (i, 0)),
            pl.BlockSpec((R, HC), lambda i: (i, 0)),
        ],
        out_shape=[
            jax.ShapeDtypeStruct((N, HC), jnp.float32),
            jax.ShapeDtypeStruct((N, HC), jnp.float32),
        ],
    )(x, W_l, b_l.reshape(1, HC), W_r, b_r.reshape(1, HC))


def _sc_edge_body(xl_hbm, xr_hbm, src_hbm, dst_hbm, att_hbm, num_hbm, den_hbm,
                  srcv, dstv, didx, xlv, xrv, denbuf, attv, acc, sem1, sem2):
    cid = lax.axis_index("c")
    sid = lax.axis_index("s")

    zero16 = jnp.zeros((16,), jnp.float32)
    lane = lax.broadcasted_iota(jnp.int32, (16,), 0)

    def zrow(r, carry):
        for k in range(D // 16):
            denbuf[r, pl.ds(16 * k, 16)] = zero16
        return carry

    lax.fori_loop(0, B + 1, zrow, 0)
    for j in range(RPT // B):
        pltpu.sync_copy(denbuf.at[pl.ds(0, B)],
                        acc.at[pl.ds(sid * RPT + j * B, B)])

    @pl.when(sid < 8)
    def _():
        pltpu.sync_copy(denbuf.at[pl.ds(0, DROWS // 8)],
                        acc.at[pl.ds(N_PAD + sid * (DROWS // 8), DROWS // 8)])

    pltpu.sync_copy(att_hbm, attv)
    attregs = [attv[pl.ds(16 * k, 16)] for k in range(HC // 16)]
    plsc.subcore_barrier()

    base_edge = (cid * NS + sid) * EPW

    def group_body(g, carry):
        e0 = g * 16
        dvec = dstv[pl.ds(e0, 16)]
        for j in range(16):
            e = e0 + j
            d = dvec[j]
            xl = [xlv[e, pl.ds(16 * k, 16)] for k in range(8)]
            xr = [xrv[e, pl.ds(16 * k, 16)] for k in range(8)]
            sub = (d & 3) * 4
            pv = jnp.zeros((16,), jnp.float32)
            for h in range(HEADS):
                s0 = xl[2 * h] + xr[2 * h]
                s1 = xl[2 * h + 1] + xr[2 * h + 1]
                t0 = jnp.maximum(s0, NEG * s0)
                t1 = jnp.maximum(s1, NEG * s1)
                u = t0 * attregs[2 * h] + t1 * attregs[2 * h + 1]
                for dshift in (8, 4, 2, 1):
                    pidx = jnp.bitwise_xor(lane, dshift)
                    u = u + u.at[pidx].get(mode="promise_in_bounds")
                ph = jnp.exp(u)
                xlv[e, pl.ds(h * 32, 16)] = ph * xl[2 * h]
                xlv[e, pl.ds(h * 32 + 16, 16)] = ph * xl[2 * h + 1]
                pv = jnp.where(lane == sub + h, ph, pv)
            for k in range(8):
                denbuf[e, pl.ds(16 * k, 16)] = zero16
            off16 = ((d & 31) >> 2) * 16
            denbuf[e, pl.ds(off16, 16)] = pv
        return carry

    def didx_body(g, carry):
        dvec = dstv[pl.ds(g * 16, 16)]
        didx[pl.ds(g * 16, 16)] = N_PAD + (dvec >> 5)
        return carry

    def chunk_body(i, carry):
        base = base_edge + i * B
        pltpu.sync_copy(src_hbm.at[pl.ds(base, B)], srcv)
        pltpu.sync_copy(dst_hbm.at[pl.ds(base, B)], dstv)
        cp1 = pltpu.async_copy(xl_hbm.at[srcv], xlv, sem1)
        cp2 = pltpu.async_copy(xr_hbm.at[dstv], xrv, sem2)
        cp1.wait()
        cp2.wait()
        lax.fori_loop(0, B // 16, didx_body, 0)
        lax.fori_loop(0, B // 16, group_body, 0)
        pltpu.sync_copy(xlv, acc.at[dstv], add=True)
        pltpu.sync_copy(denbuf.at[pl.ds(0, B)], acc.at[didx], add=True)
        return carry

    lax.fori_loop(0, NCHUNK, chunk_body, 0)
    plsc.subcore_barrier()

    pltpu.sync_copy(acc.at[pl.ds(sid * RPT, RPT)],
                    num_hbm.at[cid, pl.ds(sid * RPT, RPT)])

    @pl.when(sid < 8)
    def _():
        r0 = sid * (DROWS // 8)
        pltpu.sync_copy(acc.at[pl.ds(N_PAD + r0, DROWS // 8)],
                        den_hbm.at[cid, pl.ds(r0, DROWS // 8)])


def _sc_edge(xl, xr, src, dst, att_flat):
    mesh = plsc.VectorSubcoreMesh(core_axis_name="c", subcore_axis_name="s",
                                  num_cores=NC, num_subcores=NS)
    f = pl.kernel(
        _sc_edge_body,
        out_type=[
            jax.ShapeDtypeStruct((NC, N_PAD, D), jnp.float32),
            jax.ShapeDtypeStruct((NC, DROWS, 128), jnp.float32),
        ],
        mesh=mesh,
        scratch_types=[
            pltpu.VMEM((B,), jnp.int32),
            pltpu.VMEM((B,), jnp.int32),
            pltpu.VMEM((B,), jnp.int32),
            pltpu.VMEM((B, D), jnp.float32),
            pltpu.VMEM((B, D), jnp.float32),
            pltpu.VMEM((B + 1, 128), jnp.float32),
            pltpu.VMEM((HC,), jnp.float32),
            pltpu.VMEM_SHARED((ACC_ROWS, 128), jnp.float32),
            pltpu.SemaphoreType.DMA,
            pltpu.SemaphoreType.DMA,
        ],
    )
    return f(xl, xr, src, dst, att_flat)


def _combine_body(num_ref, den_ref, b_ref, o_ref):
    num = num_ref[0] + num_ref[1]
    dsum = den_ref[0] + den_ref[1]
    rows = num.shape[0]
    li = lax.broadcasted_iota(jnp.int32, (rows, HC), 1)
    den = jnp.zeros((rows, HC), jnp.float32)
    for h in range(HEADS):
        dh = jnp.broadcast_to(dsum[:, h:h + 1], (rows, HC))
        den = jnp.where((li // C) == h, dh, den)
    o_ref[...] = num / (den + 1e-16) + b_ref[...]


def _combine(num_partials, den_partials, bias):
    R = 2048
    return pl.pallas_call(
        _combine_body,
        grid=(N_PAD // R,),
        in_specs=[
            pl.BlockSpec((NC, R, HC), lambda i: (0, i, 0)),
            pl.BlockSpec((NC, R, HEADS), lambda i: (0, i, 0)),
            pl.BlockSpec((1, HC), lambda i: (0, 0)),
        ],
        out_specs=pl.BlockSpec((R, HC), lambda i: (i, 0)),
        out_shape=jax.ShapeDtypeStruct((N_PAD, HC), jnp.float32),
    )(num_partials, den_partials, bias.reshape(1, HC))


def kernel(x, edge_index, W_l, b_l, W_r, b_r, att, bias):
    xl, xr = _transform(x, W_l, b_l, W_r, b_r)
    src = edge_index[0]
    dst = edge_index[1]
    num_p, den_p = _sc_edge(xl, xr, src, dst, att.reshape(-1))
    den_p = den_p.reshape(NC, N_PAD, HEADS)
    return _combine(num_p, den_p, bias)[:N]
